# Initial kernel scaffold; baseline (speedup 1.0000x reference)
import jax
import jax.numpy as jnp
from jax.experimental import pallas as pl


def kernel(x, x_v, W1_w, W1_b, V_w, V_b):
    raise NotImplementedError("write your pallas kernel here")


# speedup vs baseline: 1.1101x; 1.1101x over previous
import functools
import jax
import jax.numpy as jnp
from jax.experimental import pallas as pl
from jax.experimental.pallas import tpu as pltpu

POOL = 0.125


def _score_body(x_ref, w1_ref, b1_ref, v_ref, vb_ref, s_ref):
    B, BN, D = x_ref.shape
    x = x_ref[...].reshape(B * BN, D)
    h = jnp.tanh(jax.lax.dot_general(
        x, w1_ref[...], (((1,), (1,)), ((), ())),
        preferred_element_type=jnp.float32) + b1_ref[...][None, :])
    logit = jax.lax.dot_general(
        h, v_ref[...], (((1,), (1,)), ((), ())),
        preferred_element_type=jnp.float32) + vb_ref[0]
    s_ref[...] = jax.nn.sigmoid(logit[:, 0]).reshape(B, BN)


def _pad_v(V_w):
    H = V_w.shape[1]
    return jnp.zeros((128, H), jnp.float32).at[0, :].set(V_w[0])


def _scores(x, W1_w, W1_b, V_w, V_b):
    B, N, D = x.shape
    H = W1_w.shape[0]
    BN = 2048
    grid = (N // BN,)
    return pl.pallas_call(
        _score_body,
        grid=grid,
        in_specs=[
            pl.BlockSpec((B, BN, D), lambda n: (0, n, 0)),
            pl.BlockSpec((H, D), lambda n: (0, 0)),
            pl.BlockSpec((H,), lambda n: (0,)),
            pl.BlockSpec((128, H), lambda n: (0, 0)),
            pl.BlockSpec((1,), lambda n: (0,)),
        ],
        out_specs=pl.BlockSpec((B, BN), lambda n: (0, n)),
        out_shape=jax.ShapeDtypeStruct((B, N), jnp.float32),
    )(x, W1_w, W1_b, _pad_v(V_w), V_b)


@jax.jit
def kernel(x, x_v, W1_w, W1_b, V_w, V_b):
    B, N, D = x.shape
    n = int(N * POOL)
    s = _scores(x, W1_w, W1_b, V_w, V_b)
    top_s, idx = jax.lax.top_k(s, n)
    x_g = jnp.take_along_axis(x, idx[:, :, None], axis=1)
    x_v_g = jnp.take_along_axis(x_v, idx[:, :, None], axis=1)
    return (x_g, x_v_g, top_s[:, :, None])


# speedup vs baseline: 1.1698x; 1.0538x over previous
import functools
import jax
import jax.numpy as jnp
from jax import lax
from jax.experimental import pallas as pl
from jax.experimental.pallas import tpu as pltpu
from jax.experimental.pallas import tpu_sc as plsc

POOL = 0.125
NC, NS, L = 2, 16, 16
B, N, D = 4, 8192, 128
K = int(N * POOL)
CHUNK = N // 8
NV = CHUNK // L


def _score_body(x_ref, w1_ref, b1_ref, v_ref, vb_ref, s_ref):
    Bb, BN, Dd = x_ref.shape
    x = x_ref[...].reshape(Bb * BN, Dd)
    h = jnp.tanh(lax.dot_general(
        x, w1_ref[...], (((1,), (1,)), ((), ())),
        preferred_element_type=jnp.float32) + b1_ref[...][None, :])
    logit = lax.dot_general(
        h, v_ref[...], (((1,), (1,)), ((), ())),
        preferred_element_type=jnp.float32) + vb_ref[0]
    sig = jax.nn.sigmoid(logit[:, 0]).reshape(Bb, BN)
    s_ref[...] = lax.bitcast_convert_type(sig, jnp.int32)


def _scores(x, W1_w, W1_b, V_w, V_b):
    H = W1_w.shape[0]
    BN = 2048
    v_pad = jnp.zeros((128, H), jnp.float32).at[0, :].set(V_w[0])
    return pl.pallas_call(
        _score_body,
        grid=(N // BN,),
        in_specs=[
            pl.BlockSpec((B, BN, D), lambda n: (0, n, 0)),
            pl.BlockSpec((H, D), lambda n: (0, 0)),
            pl.BlockSpec((H,), lambda n: (0,)),
            pl.BlockSpec((128, H), lambda n: (0, 0)),
            pl.BlockSpec((1,), lambda n: (0,)),
        ],
        out_specs=pl.BlockSpec((B, BN), lambda n: (0, n)),
        out_shape=jax.ShapeDtypeStruct((B, N), jnp.int32),
    )(x, W1_w, W1_b, v_pad, V_b)


_GDN = lax.GatherDimensionNumbers(
    offset_dims=(), collapsed_slice_dims=(0,), start_index_map=(0,))


def _lane_shuffle(vec, perm):
    return lax.gather(vec, perm[:, None], _GDN, (1,),
                      mode=lax.GatherScatterMode.PROMISE_IN_BOUNDS)


def _tot_gt(ak, ai, bk, bi):
    return jnp.logical_or(ak > bk, jnp.logical_and(ak == bk, ai < bi))


def _cross_stage(key_v, idx_v, k, j, npos, dir_all):
    npairs = npos // 2 // L

    def body(t, carry):
        t16 = t * L
        p = ((t16 & ~(j - 1)) << 1) | (t16 & (j - 1))
        ak = key_v[pl.ds(p, L)]
        ai = idx_v[pl.ds(p, L)]
        bk = key_v[pl.ds(p + j, L)]
        bi = idx_v[pl.ds(p + j, L)]
        aw = _tot_gt(ak, ai, bk, bi)
        if dir_all:
            ta = aw
        else:
            dv = jnp.full((L,), True) == jax.lax.broadcast((p & k) == 0, (L,))
            ta = aw == dv
        key_v[pl.ds(p, L)] = jnp.where(ta, ak, bk)
        idx_v[pl.ds(p, L)] = jnp.where(ta, ai, bi)
        key_v[pl.ds(p + j, L)] = jnp.where(ta, bk, ak)
        idx_v[pl.ds(p + j, L)] = jnp.where(ta, bi, ai)
        return carry

    lax.fori_loop(0, npairs, body, 0)


def _intra_pass(key_v, idx_v, k, js, nvec, dir_all):
    iota = lax.iota(jnp.int32, L)

    def body(v, carry):
        base = v * L
        mk = key_v[pl.ds(base, L)]
        mi = idx_v[pl.ds(base, L)]
        if dir_all:
            dm = iota >= 0
        else:
            dm = ((base + iota) & k) == 0
        for j in js:
            perm = iota ^ j
            pk = _lane_shuffle(mk, perm)
            pi = _lane_shuffle(mi, perm)
            g = _tot_gt(mk, mi, pk, pi)
            is_low = (iota & j) == 0
            keep_mine = (dm == is_low) == g
            mk = jnp.where(keep_mine, mk, pk)
            mi = jnp.where(keep_mine, mi, pi)
        key_v[pl.ds(base, L)] = mk
        idx_v[pl.ds(base, L)] = mi
        return carry

    lax.fori_loop(0, nvec, body, 0)


def _local_sort(key_v, idx_v):
    for kk in range(1, 11):
        k = 1 << kk
        js = [1 << jj for jj in range(kk - 1, -1, -1)]
        for j in [j for j in js if j >= L]:
            _cross_stage(key_v, idx_v, k, j, CHUNK, dir_all=False)
        intra = [j for j in js if j < L]
        if intra:
            _intra_pass(key_v, idx_v, k, intra, NV, dir_all=False)


def _merge_prune(key_v, idx_v, pk_v, pi_v):
    iota = lax.iota(jnp.int32, L)

    def prune(v, carry):
        base = v * L
        ak = key_v[pl.ds(base, L)]
        ai = idx_v[pl.ds(base, L)]
        rev = (CHUNK - 1 - base) - iota
        bk = plsc.load_gather(pk_v, [rev])
        bi = plsc.load_gather(pi_v, [rev])
        aw = _tot_gt(ak, ai, bk, bi)
        key_v[pl.ds(base, L)] = jnp.where(aw, ak, bk)
        idx_v[pl.ds(base, L)] = jnp.where(aw, ai, bi)
        return carry

    lax.fori_loop(0, NV, prune, 0)
    for j in (512, 256, 128, 64, 32, 16):
        _cross_stage(key_v, idx_v, 0, j, CHUNK, dir_all=True)
    _intra_pass(key_v, idx_v, 0, [8, 4, 2, 1], NV, dir_all=True)


def _sc_body(sco_hbm, x_hbm, xv_hbm, xg_hbm, xvg_hbm, sg_hbm,
             key_v, idx_v, pk_v, pi_v, idxg_v, rows_v,
             keys_sh, idx_sh, sem):
    c = lax.axis_index("c")
    s = lax.axis_index("s")
    g = s // 8
    q = s % 8
    b = c * 2 + g
    iota = lax.iota(jnp.int32, L)

    base_in = b * N + q * CHUNK
    pltpu.sync_copy(sco_hbm.at[pl.ds(base_in, CHUNK)], key_v)
    for v in range(NV):
        idx_v[pl.ds(v * L, L)] = (q * CHUNK + v * L) + iota

    _local_sort(key_v, idx_v)

    pltpu.sync_copy(key_v, keys_sh.at[s])
    pltpu.sync_copy(idx_v, idx_sh.at[s])
    plsc.subcore_barrier()

    for r in range(3):
        active = q < (4 >> r)

        @pl.when(active)
        def _read():
            slot = g * 8 + 2 * q
            pltpu.sync_copy(keys_sh.at[slot], key_v)
            pltpu.sync_copy(idx_sh.at[slot], idx_v)
            pltpu.sync_copy(keys_sh.at[slot + 1], pk_v)
            pltpu.sync_copy(idx_sh.at[slot + 1], pi_v)

        plsc.subcore_barrier()

        @pl.when(active)
        def _merge():
            _merge_prune(key_v, idx_v, pk_v, pi_v)
            pltpu.sync_copy(key_v, keys_sh.at[g * 8 + q])
            pltpu.sync_copy(idx_v, idx_sh.at[g * 8 + q])

        plsc.subcore_barrier()

    @pl.when(q == 0)
    def _write_scores():
        pltpu.sync_copy(key_v, sg_hbm.at[pl.ds(b * K, K)])

    rows_per_tile = K // 8
    pltpu.sync_copy(idx_sh.at[g * 8, pl.ds(q * rows_per_tile, rows_per_tile)],
                    idxg_v)
    for t in range(rows_per_tile // L):
        idxg_v[pl.ds(t * L, L)] = idxg_v[pl.ds(t * L, L)] + b * N
    out_base = b * K + q * rows_per_tile
    pltpu.async_copy(x_hbm.at[idxg_v], rows_v, sem).wait()
    pltpu.sync_copy(rows_v, xg_hbm.at[pl.ds(out_base, rows_per_tile)])
    pltpu.async_copy(xv_hbm.at[idxg_v], rows_v, sem).wait()
    pltpu.sync_copy(rows_v, xvg_hbm.at[pl.ds(out_base, rows_per_tile)])


_sc_topk = functools.partial(
    pl.kernel,
    out_type=(
        jax.ShapeDtypeStruct((B * K, D), jnp.float32),
        jax.ShapeDtypeStruct((B * K, D), jnp.float32),
        jax.ShapeDtypeStruct((B * K,), jnp.int32),
    ),
    mesh=plsc.VectorSubcoreMesh(core_axis_name="c", subcore_axis_name="s"),
    compiler_params=pltpu.CompilerParams(needs_layout_passes=False),
    scratch_types=[
        pltpu.VMEM((CHUNK,), jnp.int32),
        pltpu.VMEM((CHUNK,), jnp.int32),
        pltpu.VMEM((CHUNK,), jnp.int32),
        pltpu.VMEM((CHUNK,), jnp.int32),
        pltpu.VMEM((K // 8,), jnp.int32),
        pltpu.VMEM((K // 8, D), jnp.float32),
        pltpu.VMEM_SHARED((NS, CHUNK), jnp.int32),
        pltpu.VMEM_SHARED((NS, CHUNK), jnp.int32),
        pltpu.SemaphoreType.DMA,
    ],
)(_sc_body)


@jax.jit
def kernel(x, x_v, W1_w, W1_b, V_w, V_b):
    s = _scores(x, W1_w, W1_b, V_w, V_b)
    xg, xvg, sg = _sc_topk(s.reshape(B * N),
                           x.reshape(B * N, D),
                           x_v.reshape(B * N, D))
    sgf = lax.bitcast_convert_type(sg, jnp.float32)
    return (xg.reshape(B, K, D), xvg.reshape(B, K, D),
            sgf.reshape(B, K, 1))


# speedup vs baseline: 1.4740x; 1.2601x over previous
import functools
import jax
import jax.numpy as jnp
from jax import lax
from jax.experimental import pallas as pl
from jax.experimental.pallas import tpu as pltpu
from jax.experimental.pallas import tpu_sc as plsc

POOL = 0.125
NC, NS, L = 2, 16, 16
B, N, D = 4, 8192, 128
K = int(N * POOL)
CHUNK = N // 8
NV = CHUNK // L


def _score_body(x_ref, w1_ref, b1_ref, v_ref, vb_ref, s_ref):
    Bb, BN, Dd = x_ref.shape
    x = x_ref[...].reshape(Bb * BN, Dd)
    h = jnp.tanh(lax.dot_general(
        x, w1_ref[...], (((1,), (1,)), ((), ())),
        preferred_element_type=jnp.float32) + b1_ref[...][None, :])
    logit = lax.dot_general(
        h, v_ref[...], (((1,), (1,)), ((), ())),
        preferred_element_type=jnp.float32) + vb_ref[0]
    sig = jax.nn.sigmoid(logit[:, 0]).reshape(Bb, BN)
    s_ref[...] = lax.bitcast_convert_type(sig, jnp.int32)


def _scores(x, W1_w, W1_b, V_w, V_b):
    H = W1_w.shape[0]
    BN = 2048
    v_pad = jnp.zeros((128, H), jnp.float32).at[0, :].set(V_w[0])
    return pl.pallas_call(
        _score_body,
        grid=(N // BN,),
        in_specs=[
            pl.BlockSpec((B, BN, D), lambda n: (0, n, 0)),
            pl.BlockSpec((H, D), lambda n: (0, 0)),
            pl.BlockSpec((H,), lambda n: (0,)),
            pl.BlockSpec((128, H), lambda n: (0, 0)),
            pl.BlockSpec((1,), lambda n: (0,)),
        ],
        out_specs=pl.BlockSpec((B, BN), lambda n: (0, n)),
        out_shape=jax.ShapeDtypeStruct((B, N), jnp.int32),
    )(x, W1_w, W1_b, v_pad, V_b)


_GDN = lax.GatherDimensionNumbers(
    offset_dims=(), collapsed_slice_dims=(0,), start_index_map=(0,))


def _lane_shuffle(vec, perm):
    return lax.gather(vec, perm[:, None], _GDN, (1,),
                      mode=lax.GatherScatterMode.PROMISE_IN_BOUNDS)


def _tot_gt(ak, ai, bk, bi):
    return jnp.logical_or(ak > bk, jnp.logical_and(ak == bk, ai < bi))


def _cross_stage(key_v, idx_v, k, j, npos, dir_all):
    npairs = npos // 2 // L

    @plsc.parallel_loop(0, npairs, unroll=4)
    def _(t):
        t16 = t * L
        p = ((t16 & ~(j - 1)) << 1) | (t16 & (j - 1))
        ak = key_v[pl.ds(p, L)]
        ai = idx_v[pl.ds(p, L)]
        bk = key_v[pl.ds(p + j, L)]
        bi = idx_v[pl.ds(p + j, L)]
        aw = _tot_gt(ak, ai, bk, bi)
        if dir_all:
            ta = aw
        else:
            dv = jnp.full((L,), True) == jax.lax.broadcast((p & k) == 0, (L,))
            ta = aw == dv
        key_v[pl.ds(p, L)] = jnp.where(ta, ak, bk)
        idx_v[pl.ds(p, L)] = jnp.where(ta, ai, bi)
        key_v[pl.ds(p + j, L)] = jnp.where(ta, bk, ak)
        idx_v[pl.ds(p + j, L)] = jnp.where(ta, bi, ai)


def _intra_pass(key_v, idx_v, k, js, nvec, dir_all):
    iota = lax.iota(jnp.int32, L)

    @plsc.parallel_loop(0, nvec, unroll=2)
    def _(v):
        base = v * L
        mk = key_v[pl.ds(base, L)]
        mi = idx_v[pl.ds(base, L)]
        if dir_all:
            dm = iota >= 0
        else:
            dm = ((base + iota) & k) == 0
        for j in js:
            perm = iota ^ j
            pk = _lane_shuffle(mk, perm)
            pi = _lane_shuffle(mi, perm)
            g = _tot_gt(mk, mi, pk, pi)
            is_low = (iota & j) == 0
            keep_mine = (dm == is_low) == g
            mk = jnp.where(keep_mine, mk, pk)
            mi = jnp.where(keep_mine, mi, pi)
        key_v[pl.ds(base, L)] = mk
        idx_v[pl.ds(base, L)] = mi


def _local_sort(key_v, idx_v):
    for kk in range(1, 11):
        k = 1 << kk
        js = [1 << jj for jj in range(kk - 1, -1, -1)]
        for j in [j for j in js if j >= L]:
            _cross_stage(key_v, idx_v, k, j, CHUNK, dir_all=False)
        intra = [j for j in js if j < L]
        if intra:
            _intra_pass(key_v, idx_v, k, intra, NV, dir_all=False)


def _merge_prune(key_v, idx_v, pk_v, pi_v):
    iota = lax.iota(jnp.int32, L)

    @plsc.parallel_loop(0, NV, unroll=4)
    def _(v):
        base = v * L
        ak = key_v[pl.ds(base, L)]
        ai = idx_v[pl.ds(base, L)]
        rev = (CHUNK - 1 - base) - iota
        bk = plsc.load_gather(pk_v, [rev])
        bi = plsc.load_gather(pi_v, [rev])
        aw = _tot_gt(ak, ai, bk, bi)
        key_v[pl.ds(base, L)] = jnp.where(aw, ak, bk)
        idx_v[pl.ds(base, L)] = jnp.where(aw, ai, bi)
    for j in (512, 256, 128, 64, 32, 16):
        _cross_stage(key_v, idx_v, 0, j, CHUNK, dir_all=True)
    _intra_pass(key_v, idx_v, 0, [8, 4, 2, 1], NV, dir_all=True)


def _sc_body(sco_hbm, x_hbm, xv_hbm, xg_hbm, xvg_hbm, sg_hbm,
             key_v, idx_v, pk_v, pi_v, idxg_v, rows_v,
             keys_sh, idx_sh, sem):
    c = lax.axis_index("c")
    s = lax.axis_index("s")
    g = s // 8
    q = s % 8
    b = c * 2 + g
    iota = lax.iota(jnp.int32, L)

    base_in = b * N + q * CHUNK
    pltpu.sync_copy(sco_hbm.at[pl.ds(base_in, CHUNK)], key_v)
    for v in range(NV):
        idx_v[pl.ds(v * L, L)] = (q * CHUNK + v * L) + iota

    _local_sort(key_v, idx_v)

    pltpu.sync_copy(key_v, keys_sh.at[s])
    pltpu.sync_copy(idx_v, idx_sh.at[s])
    plsc.subcore_barrier()

    for r in range(3):
        active = q < (4 >> r)

        @pl.when(active)
        def _read():
            slot = g * 8 + 2 * q
            pltpu.sync_copy(keys_sh.at[slot], key_v)
            pltpu.sync_copy(idx_sh.at[slot], idx_v)
            pltpu.sync_copy(keys_sh.at[slot + 1], pk_v)
            pltpu.sync_copy(idx_sh.at[slot + 1], pi_v)

        plsc.subcore_barrier()

        @pl.when(active)
        def _merge():
            _merge_prune(key_v, idx_v, pk_v, pi_v)
            pltpu.sync_copy(key_v, keys_sh.at[g * 8 + q])
            pltpu.sync_copy(idx_v, idx_sh.at[g * 8 + q])

        plsc.subcore_barrier()

    @pl.when(q == 0)
    def _write_scores():
        pltpu.sync_copy(key_v, sg_hbm.at[pl.ds(b * K, K)])

    rows_per_tile = K // 8
    pltpu.sync_copy(idx_sh.at[g * 8, pl.ds(q * rows_per_tile, rows_per_tile)],
                    idxg_v)
    for t in range(rows_per_tile // L):
        idxg_v[pl.ds(t * L, L)] = idxg_v[pl.ds(t * L, L)] + b * N
    out_base = b * K + q * rows_per_tile
    pltpu.async_copy(x_hbm.at[idxg_v], rows_v, sem).wait()
    pltpu.sync_copy(rows_v, xg_hbm.at[pl.ds(out_base, rows_per_tile)])
    pltpu.async_copy(xv_hbm.at[idxg_v], rows_v, sem).wait()
    pltpu.sync_copy(rows_v, xvg_hbm.at[pl.ds(out_base, rows_per_tile)])


_sc_topk = functools.partial(
    pl.kernel,
    out_type=(
        jax.ShapeDtypeStruct((B * K, D), jnp.float32),
        jax.ShapeDtypeStruct((B * K, D), jnp.float32),
        jax.ShapeDtypeStruct((B * K,), jnp.int32),
    ),
    mesh=plsc.VectorSubcoreMesh(core_axis_name="c", subcore_axis_name="s"),
    compiler_params=pltpu.CompilerParams(needs_layout_passes=False),
    scratch_types=[
        pltpu.VMEM((CHUNK,), jnp.int32),
        pltpu.VMEM((CHUNK,), jnp.int32),
        pltpu.VMEM((CHUNK,), jnp.int32),
        pltpu.VMEM((CHUNK,), jnp.int32),
        pltpu.VMEM((K // 8,), jnp.int32),
        pltpu.VMEM((K // 8, D), jnp.float32),
        pltpu.VMEM_SHARED((NS, CHUNK), jnp.int32),
        pltpu.VMEM_SHARED((NS, CHUNK), jnp.int32),
        pltpu.SemaphoreType.DMA,
    ],
)(_sc_body)


@jax.jit
def kernel(x, x_v, W1_w, W1_b, V_w, V_b):
    s = _scores(x, W1_w, W1_b, V_w, V_b)
    xg, xvg, sg = _sc_topk(s.reshape(B * N),
                           x.reshape(B * N, D),
                           x_v.reshape(B * N, D))
    sgf = lax.bitcast_convert_type(sg, jnp.float32)
    return (xg.reshape(B, K, D), xvg.reshape(B, K, D),
            sgf.reshape(B, K, 1))


# speedup vs baseline: 1.5116x; 1.0255x over previous
import functools
import jax
import jax.numpy as jnp
from jax import lax
from jax.experimental import pallas as pl
from jax.experimental.pallas import tpu as pltpu
from jax.experimental.pallas import tpu_sc as plsc

POOL = 0.125
NC, NS, L = 2, 16, 16
B, N, D = 4, 8192, 128
K = int(N * POOL)
CHUNK = N // 8
NV = CHUNK // L


def _score_body(x_ref, w1_ref, b1_ref, v_ref, vb_ref, s_ref):
    Bb, BN, Dd = x_ref.shape
    x = x_ref[...].reshape(Bb * BN, Dd)
    h = jnp.tanh(lax.dot_general(
        x, w1_ref[...], (((1,), (1,)), ((), ())),
        preferred_element_type=jnp.float32) + b1_ref[...][None, :])
    logit = lax.dot_general(
        h, v_ref[...], (((1,), (1,)), ((), ())),
        preferred_element_type=jnp.float32) + vb_ref[0]
    sig = jax.nn.sigmoid(logit[:, 0]).reshape(Bb, BN)
    s_ref[...] = lax.bitcast_convert_type(sig, jnp.int32)


def _scores(x, W1_w, W1_b, V_w, V_b):
    H = W1_w.shape[0]
    BN = 2048
    v_pad = jnp.zeros((128, H), jnp.float32).at[0, :].set(V_w[0])
    return pl.pallas_call(
        _score_body,
        grid=(N // BN,),
        in_specs=[
            pl.BlockSpec((B, BN, D), lambda n: (0, n, 0)),
            pl.BlockSpec((H, D), lambda n: (0, 0)),
            pl.BlockSpec((H,), lambda n: (0,)),
            pl.BlockSpec((128, H), lambda n: (0, 0)),
            pl.BlockSpec((1,), lambda n: (0,)),
        ],
        out_specs=pl.BlockSpec((B, BN), lambda n: (0, n)),
        out_shape=jax.ShapeDtypeStruct((B, N), jnp.int32),
    )(x, W1_w, W1_b, v_pad, V_b)


_GDN = lax.GatherDimensionNumbers(
    offset_dims=(), collapsed_slice_dims=(0,), start_index_map=(0,))


def _lane_shuffle(vec, perm):
    return lax.gather(vec, perm[:, None], _GDN, (1,),
                      mode=lax.GatherScatterMode.PROMISE_IN_BOUNDS)


def _tot_gt(ak, ai, bk, bi):
    return jnp.logical_or(ak > bk, jnp.logical_and(ak == bk, ai < bi))


def _cross_stage(key_v, idx_v, k, j, npos, dir_all):
    npairs = npos // 2 // L

    @plsc.parallel_loop(0, npairs, unroll=2)
    def _(t):
        t16 = t * L
        jm1 = j - 1
        p = ((t16 & ~jm1) << 1) | (t16 & jm1)
        ak = key_v[pl.ds(p, L)]
        ai = idx_v[pl.ds(p, L)]
        bk = key_v[pl.ds(p + j, L)]
        bi = idx_v[pl.ds(p + j, L)]
        aw = _tot_gt(ak, ai, bk, bi)
        if dir_all:
            ta = aw
        else:
            dv = jnp.full((L,), True) == jax.lax.broadcast((p & k) == 0, (L,))
            ta = aw == dv
        key_v[pl.ds(p, L)] = jnp.where(ta, ak, bk)
        idx_v[pl.ds(p, L)] = jnp.where(ta, ai, bi)
        key_v[pl.ds(p + j, L)] = jnp.where(ta, bk, ak)
        idx_v[pl.ds(p + j, L)] = jnp.where(ta, bi, ai)


def _intra_pass(key_v, idx_v, k, nvec, dir_all):
    iota = lax.iota(jnp.int32, L)

    @plsc.parallel_loop(0, nvec, unroll=2)
    def _(v):
        base = v * L
        mk = key_v[pl.ds(base, L)]
        mi = idx_v[pl.ds(base, L)]
        if dir_all:
            dm = iota >= 0
        else:
            dm = ((base + iota) & k) == 0
        for j in (8, 4, 2, 1):
            perm = iota ^ j
            pk = _lane_shuffle(mk, perm)
            pi = _lane_shuffle(mi, perm)
            g = _tot_gt(mk, mi, pk, pi)
            is_low = (iota & j) == 0
            keep_mine = (dm == is_low) == g
            if not dir_all:
                off = jax.lax.broadcast(j >= k, (L,))
                keep_mine = jnp.logical_or(keep_mine, off)
            mk = jnp.where(keep_mine, mk, pk)
            mi = jnp.where(keep_mine, mi, pi)
        key_v[pl.ds(base, L)] = mk
        idx_v[pl.ds(base, L)] = mi


def _local_sort(key_v, idx_v):
    def level(kk, carry):
        k = 1 << kk

        def cross(i, carry2):
            _cross_stage(key_v, idx_v, k, k >> (i + 1), CHUNK, dir_all=False)
            return carry2

        lax.fori_loop(0, jnp.maximum(kk - 4, 0), cross, 0)
        _intra_pass(key_v, idx_v, k, NV, dir_all=False)
        return carry

    lax.fori_loop(1, 11, level, 0)


def _merge_prune(key_v, idx_v, pk_v, pi_v):
    iota = lax.iota(jnp.int32, L)

    @plsc.parallel_loop(0, NV, unroll=2)
    def _(v):
        base = v * L
        ak = key_v[pl.ds(base, L)]
        ai = idx_v[pl.ds(base, L)]
        rev = (CHUNK - 1 - base) - iota
        bk = plsc.load_gather(pk_v, [rev])
        bi = plsc.load_gather(pi_v, [rev])
        aw = _tot_gt(ak, ai, bk, bi)
        key_v[pl.ds(base, L)] = jnp.where(aw, ak, bk)
        idx_v[pl.ds(base, L)] = jnp.where(aw, ai, bi)

    def cross(i, carry):
        _cross_stage(key_v, idx_v, 0, 512 >> i, CHUNK, dir_all=True)
        return carry

    lax.fori_loop(0, 6, cross, 0)
    _intra_pass(key_v, idx_v, 0, NV, dir_all=True)


def _sc_body(sco_hbm, x_hbm, xv_hbm, xg_hbm, xvg_hbm, sg_hbm,
             key_v, idx_v, pk_v, pi_v, idxg_v, rows_v,
             keys_sh, idx_sh, sem):
    c = lax.axis_index("c")
    s = lax.axis_index("s")
    g = s // 8
    q = s % 8
    b = c * 2 + g
    iota = lax.iota(jnp.int32, L)

    base_in = b * N + q * CHUNK
    pltpu.sync_copy(sco_hbm.at[pl.ds(base_in, CHUNK)], key_v)

    @plsc.parallel_loop(0, NV, unroll=2)
    def _(v):
        idx_v[pl.ds(v * L, L)] = (q * CHUNK + v * L) + iota

    _local_sort(key_v, idx_v)

    pltpu.sync_copy(key_v, keys_sh.at[s])
    pltpu.sync_copy(idx_v, idx_sh.at[s])
    plsc.subcore_barrier()

    for r in range(3):
        active = q < (4 >> r)

        @pl.when(active)
        def _read():
            slot = g * 8 + 2 * q
            pltpu.sync_copy(keys_sh.at[slot], key_v)
            pltpu.sync_copy(idx_sh.at[slot], idx_v)
            pltpu.sync_copy(keys_sh.at[slot + 1], pk_v)
            pltpu.sync_copy(idx_sh.at[slot + 1], pi_v)

        plsc.subcore_barrier()

        @pl.when(active)
        def _merge():
            _merge_prune(key_v, idx_v, pk_v, pi_v)
            pltpu.sync_copy(key_v, keys_sh.at[g * 8 + q])
            pltpu.sync_copy(idx_v, idx_sh.at[g * 8 + q])

        plsc.subcore_barrier()

    @pl.when(q == 0)
    def _write_scores():
        pltpu.sync_copy(key_v, sg_hbm.at[pl.ds(b * K, K)])

    rows_per_tile = K // 8
    pltpu.sync_copy(idx_sh.at[g * 8, pl.ds(q * rows_per_tile, rows_per_tile)],
                    idxg_v)
    for t in range(rows_per_tile // L):
        idxg_v[pl.ds(t * L, L)] = idxg_v[pl.ds(t * L, L)] + b * N
    out_base = b * K + q * rows_per_tile
    pltpu.async_copy(x_hbm.at[idxg_v], rows_v, sem).wait()
    pltpu.sync_copy(rows_v, xg_hbm.at[pl.ds(out_base, rows_per_tile)])
    pltpu.async_copy(xv_hbm.at[idxg_v], rows_v, sem).wait()
    pltpu.sync_copy(rows_v, xvg_hbm.at[pl.ds(out_base, rows_per_tile)])


_sc_topk = functools.partial(
    pl.kernel,
    out_type=(
        jax.ShapeDtypeStruct((B * K, D), jnp.float32),
        jax.ShapeDtypeStruct((B * K, D), jnp.float32),
        jax.ShapeDtypeStruct((B * K,), jnp.int32),
    ),
    mesh=plsc.VectorSubcoreMesh(core_axis_name="c", subcore_axis_name="s"),
    compiler_params=pltpu.CompilerParams(needs_layout_passes=False),
    scratch_types=[
        pltpu.VMEM((CHUNK,), jnp.int32),
        pltpu.VMEM((CHUNK,), jnp.int32),
        pltpu.VMEM((CHUNK,), jnp.int32),
        pltpu.VMEM((CHUNK,), jnp.int32),
        pltpu.VMEM((K // 8,), jnp.int32),
        pltpu.VMEM((K // 8, D), jnp.float32),
        pltpu.VMEM_SHARED((NS, CHUNK), jnp.int32),
        pltpu.VMEM_SHARED((NS, CHUNK), jnp.int32),
        pltpu.SemaphoreType.DMA,
    ],
)(_sc_body)


@jax.jit
def kernel(x, x_v, W1_w, W1_b, V_w, V_b):
    s = _scores(x, W1_w, W1_b, V_w, V_b)
    xg, xvg, sg = _sc_topk(s.reshape(B * N),
                           x.reshape(B * N, D),
                           x_v.reshape(B * N, D))
    sgf = lax.bitcast_convert_type(sg, jnp.float32)
    return (xg.reshape(B, K, D), xvg.reshape(B, K, D),
            sgf.reshape(B, K, 1))


# speedup vs baseline: 1.5618x; 1.0332x over previous
import functools
import jax
import jax.numpy as jnp
from jax import lax
from jax.experimental import pallas as pl
from jax.experimental.pallas import tpu as pltpu
from jax.experimental.pallas import tpu_sc as plsc

POOL = 0.125
NC, NS, L = 2, 16, 16
B, N, D = 4, 8192, 128
K = int(N * POOL)
CHUNK = N // 8
NV = CHUNK // L


def _score_body(x_ref, w1_ref, b1_ref, v_ref, vb_ref, s_ref):
    Bb, BN, Dd = x_ref.shape
    x = x_ref[...].reshape(Bb * BN, Dd)
    h = jnp.tanh(lax.dot_general(
        x, w1_ref[...], (((1,), (1,)), ((), ())),
        preferred_element_type=jnp.float32) + b1_ref[...][None, :])
    logit = lax.dot_general(
        h, v_ref[...], (((1,), (1,)), ((), ())),
        preferred_element_type=jnp.float32) + vb_ref[0]
    sig = jax.nn.sigmoid(logit[:, 0]).reshape(Bb, BN)
    s_ref[...] = lax.bitcast_convert_type(sig, jnp.int32)


def _scores(x, W1_w, W1_b, V_w, V_b):
    H = W1_w.shape[0]
    BN = 2048
    v_pad = jnp.zeros((128, H), jnp.float32).at[0, :].set(V_w[0])
    return pl.pallas_call(
        _score_body,
        grid=(N // BN,),
        in_specs=[
            pl.BlockSpec((B, BN, D), lambda n: (0, n, 0)),
            pl.BlockSpec((H, D), lambda n: (0, 0)),
            pl.BlockSpec((H,), lambda n: (0,)),
            pl.BlockSpec((128, H), lambda n: (0, 0)),
            pl.BlockSpec((1,), lambda n: (0,)),
        ],
        out_specs=pl.BlockSpec((B, BN), lambda n: (0, n)),
        out_shape=jax.ShapeDtypeStruct((B, N), jnp.int32),
    )(x, W1_w, W1_b, v_pad, V_b)


_GDN = lax.GatherDimensionNumbers(
    offset_dims=(), collapsed_slice_dims=(0,), start_index_map=(0,))


def _lane_shuffle(vec, perm):
    return lax.gather(vec, perm[:, None], _GDN, (1,),
                      mode=lax.GatherScatterMode.PROMISE_IN_BOUNDS)


def _tot_gt(ak, ai, bk, bi):
    return jnp.logical_or(ak > bk, jnp.logical_and(ak == bk, ai < bi))


def _cross_stage(key_v, idx_v, k, j, npos, dir_all):
    npairs = npos // 2 // L

    @plsc.parallel_loop(0, npairs, unroll=2)
    def _(t):
        t16 = t * L
        jm1 = j - 1
        p = ((t16 & ~jm1) << 1) | (t16 & jm1)
        ak = key_v[pl.ds(p, L)]
        ai = idx_v[pl.ds(p, L)]
        bk = key_v[pl.ds(p + j, L)]
        bi = idx_v[pl.ds(p + j, L)]
        aw = _tot_gt(ak, ai, bk, bi)
        if dir_all:
            ta = aw
        else:
            dv = jnp.full((L,), True) == jax.lax.broadcast((p & k) == 0, (L,))
            ta = aw == dv
        key_v[pl.ds(p, L)] = jnp.where(ta, ak, bk)
        idx_v[pl.ds(p, L)] = jnp.where(ta, ai, bi)
        key_v[pl.ds(p + j, L)] = jnp.where(ta, bk, ak)
        idx_v[pl.ds(p + j, L)] = jnp.where(ta, bi, ai)


def _intra_pass(key_v, idx_v, k, nvec, dir_all):
    iota = lax.iota(jnp.int32, L)

    @plsc.parallel_loop(0, nvec, unroll=2)
    def _(v):
        base = v * L
        mk = key_v[pl.ds(base, L)]
        mi = idx_v[pl.ds(base, L)]
        if dir_all:
            dm = iota >= 0
        else:
            dm = ((base + iota) & k) == 0
        for j in (8, 4, 2, 1):
            perm = iota ^ j
            pk = _lane_shuffle(mk, perm)
            pi = _lane_shuffle(mi, perm)
            g = _tot_gt(mk, mi, pk, pi)
            is_low = (iota & j) == 0
            keep_mine = (dm == is_low) == g
            mk = jnp.where(keep_mine, mk, pk)
            mi = jnp.where(keep_mine, mi, pi)
        key_v[pl.ds(base, L)] = mk
        idx_v[pl.ds(base, L)] = mi


def _vreg_sort16(key_v, idx_v):
    iota = lax.iota(jnp.int32, L)
    stages = [(2, 1), (4, 2), (4, 1), (8, 4), (8, 2), (8, 1),
              (16, 8), (16, 4), (16, 2), (16, 1)]

    @plsc.parallel_loop(0, NV, unroll=2)
    def _(v):
        base = v * L
        mk = key_v[pl.ds(base, L)]
        mi = idx_v[pl.ds(base, L)]
        for k, j in stages:
            dm = ((base + iota) & k) == 0
            perm = iota ^ j
            pk = _lane_shuffle(mk, perm)
            pi = _lane_shuffle(mi, perm)
            g = _tot_gt(mk, mi, pk, pi)
            is_low = (iota & j) == 0
            keep_mine = (dm == is_low) == g
            mk = jnp.where(keep_mine, mk, pk)
            mi = jnp.where(keep_mine, mi, pi)
        key_v[pl.ds(base, L)] = mk
        idx_v[pl.ds(base, L)] = mi


def _local_sort(key_v, idx_v):
    _vreg_sort16(key_v, idx_v)

    def level(kk, carry):
        k = 1 << kk

        def cross(i, carry2):
            _cross_stage(key_v, idx_v, k, k >> (i + 1), CHUNK, dir_all=False)
            return carry2

        lax.fori_loop(0, kk - 4, cross, 0)
        _intra_pass(key_v, idx_v, k, NV, dir_all=False)
        return carry

    lax.fori_loop(5, 11, level, 0)


def _merge_prune(key_v, idx_v, pk_v, pi_v):
    iota = lax.iota(jnp.int32, L)

    @plsc.parallel_loop(0, NV, unroll=2)
    def _(v):
        base = v * L
        ak = key_v[pl.ds(base, L)]
        ai = idx_v[pl.ds(base, L)]
        rev = (CHUNK - 1 - base) - iota
        bk = plsc.load_gather(pk_v, [rev])
        bi = plsc.load_gather(pi_v, [rev])
        aw = _tot_gt(ak, ai, bk, bi)
        key_v[pl.ds(base, L)] = jnp.where(aw, ak, bk)
        idx_v[pl.ds(base, L)] = jnp.where(aw, ai, bi)

    def cross(i, carry):
        _cross_stage(key_v, idx_v, 0, 512 >> i, CHUNK, dir_all=True)
        return carry

    lax.fori_loop(0, 6, cross, 0)
    _intra_pass(key_v, idx_v, 0, NV, dir_all=True)


def _sc_body(sco_hbm, x_hbm, xv_hbm, xg_hbm, xvg_hbm, sg_hbm,
             key_v, idx_v, pk_v, pi_v, idxg_v, rows_v,
             keys_sh, idx_sh, sem):
    c = lax.axis_index("c")
    s = lax.axis_index("s")
    g = s // 8
    q = s % 8
    b = c * 2 + g
    iota = lax.iota(jnp.int32, L)

    base_in = b * N + q * CHUNK
    pltpu.sync_copy(sco_hbm.at[pl.ds(base_in, CHUNK)], key_v)

    @plsc.parallel_loop(0, NV, unroll=2)
    def _(v):
        idx_v[pl.ds(v * L, L)] = (q * CHUNK + v * L) + iota

    _local_sort(key_v, idx_v)

    pltpu.sync_copy(key_v, keys_sh.at[s])
    pltpu.sync_copy(idx_v, idx_sh.at[s])
    plsc.subcore_barrier()

    for r in range(3):
        active = q < (4 >> r)

        @pl.when(active)
        def _read():
            slot = g * 8 + 2 * q
            pltpu.sync_copy(keys_sh.at[slot], key_v)
            pltpu.sync_copy(idx_sh.at[slot], idx_v)
            pltpu.sync_copy(keys_sh.at[slot + 1], pk_v)
            pltpu.sync_copy(idx_sh.at[slot + 1], pi_v)

        plsc.subcore_barrier()

        @pl.when(active)
        def _merge():
            _merge_prune(key_v, idx_v, pk_v, pi_v)
            pltpu.sync_copy(key_v, keys_sh.at[g * 8 + q])
            pltpu.sync_copy(idx_v, idx_sh.at[g * 8 + q])

        plsc.subcore_barrier()

    @pl.when(q == 0)
    def _write_scores():
        pltpu.sync_copy(key_v, sg_hbm.at[pl.ds(b * K, K)])

    rows_per_tile = K // 8
    pltpu.sync_copy(idx_sh.at[g * 8, pl.ds(q * rows_per_tile, rows_per_tile)],
                    idxg_v)
    for t in range(rows_per_tile // L):
        idxg_v[pl.ds(t * L, L)] = idxg_v[pl.ds(t * L, L)] + b * N
    out_base = b * K + q * rows_per_tile
    pltpu.async_copy(x_hbm.at[idxg_v], rows_v, sem).wait()
    pltpu.sync_copy(rows_v, xg_hbm.at[pl.ds(out_base, rows_per_tile)])
    pltpu.async_copy(xv_hbm.at[idxg_v], rows_v, sem).wait()
    pltpu.sync_copy(rows_v, xvg_hbm.at[pl.ds(out_base, rows_per_tile)])


_sc_topk = functools.partial(
    pl.kernel,
    out_type=(
        jax.ShapeDtypeStruct((B * K, D), jnp.float32),
        jax.ShapeDtypeStruct((B * K, D), jnp.float32),
        jax.ShapeDtypeStruct((B * K,), jnp.int32),
    ),
    mesh=plsc.VectorSubcoreMesh(core_axis_name="c", subcore_axis_name="s"),
    compiler_params=pltpu.CompilerParams(needs_layout_passes=False),
    scratch_types=[
        pltpu.VMEM((CHUNK,), jnp.int32),
        pltpu.VMEM((CHUNK,), jnp.int32),
        pltpu.VMEM((CHUNK,), jnp.int32),
        pltpu.VMEM((CHUNK,), jnp.int32),
        pltpu.VMEM((K // 8,), jnp.int32),
        pltpu.VMEM((K // 8, D), jnp.float32),
        pltpu.VMEM_SHARED((NS, CHUNK), jnp.int32),
        pltpu.VMEM_SHARED((NS, CHUNK), jnp.int32),
        pltpu.SemaphoreType.DMA,
    ],
)(_sc_body)


@jax.jit
def kernel(x, x_v, W1_w, W1_b, V_w, V_b):
    s = _scores(x, W1_w, W1_b, V_w, V_b)
    xg, xvg, sg = _sc_topk(s.reshape(B * N),
                           x.reshape(B * N, D),
                           x_v.reshape(B * N, D))
    sgf = lax.bitcast_convert_type(sg, jnp.float32)
    return (xg.reshape(B, K, D), xvg.reshape(B, K, D),
            sgf.reshape(B, K, 1))


# speedup vs baseline: 1.5945x; 1.0209x over previous
import functools
import jax
import jax.numpy as jnp
from jax import lax
from jax.experimental import pallas as pl
from jax.experimental.pallas import tpu as pltpu
from jax.experimental.pallas import tpu_sc as plsc

POOL = 0.125
NC, NS, L = 2, 16, 16
B, N, D = 4, 8192, 128
K = int(N * POOL)
CHUNK = N // 8
NV = CHUNK // L


def _score_body(x_ref, w1_ref, b1_ref, v_ref, vb_ref, s_ref):
    Bb, BN, Dd = x_ref.shape
    x = x_ref[...].reshape(Bb * BN, Dd)
    h = jnp.tanh(lax.dot_general(
        x, w1_ref[...], (((1,), (1,)), ((), ())),
        preferred_element_type=jnp.float32) + b1_ref[...][None, :])
    logit = lax.dot_general(
        h, v_ref[...], (((1,), (1,)), ((), ())),
        preferred_element_type=jnp.float32) + vb_ref[0]
    sig = jax.nn.sigmoid(logit[:, 0]).reshape(Bb, BN)
    s_ref[...] = lax.bitcast_convert_type(sig, jnp.int32)


def _scores(x, W1_w, W1_b, V_w, V_b):
    H = W1_w.shape[0]
    BN = 2048
    v_pad = jnp.zeros((128, H), jnp.float32).at[0, :].set(V_w[0])
    return pl.pallas_call(
        _score_body,
        grid=(N // BN,),
        in_specs=[
            pl.BlockSpec((B, BN, D), lambda n: (0, n, 0)),
            pl.BlockSpec((H, D), lambda n: (0, 0)),
            pl.BlockSpec((H,), lambda n: (0,)),
            pl.BlockSpec((128, H), lambda n: (0, 0)),
            pl.BlockSpec((1,), lambda n: (0,)),
        ],
        out_specs=pl.BlockSpec((B, BN), lambda n: (0, n)),
        out_shape=jax.ShapeDtypeStruct((B, N), jnp.int32),
    )(x, W1_w, W1_b, v_pad, V_b)


_GDN = lax.GatherDimensionNumbers(
    offset_dims=(), collapsed_slice_dims=(0,), start_index_map=(0,))


def _lane_shuffle(vec, perm):
    return lax.gather(vec, perm[:, None], _GDN, (1,),
                      mode=lax.GatherScatterMode.PROMISE_IN_BOUNDS)


def _tot_gt(ak, ai, bk, bi):
    return jnp.logical_or(ak > bk, jnp.logical_and(ak == bk, ai < bi))


def _cross_stage(key_v, idx_v, k, j, npos, dir_all, start=0):
    npairs = npos // 2 // L

    @plsc.parallel_loop(0, npairs, unroll=2)
    def _(t):
        t16 = t * L
        jm1 = j - 1
        p = start + (((t16 & ~jm1) << 1) | (t16 & jm1))
        ak = key_v[pl.ds(p, L)]
        ai = idx_v[pl.ds(p, L)]
        bk = key_v[pl.ds(p + j, L)]
        bi = idx_v[pl.ds(p + j, L)]
        aw = _tot_gt(ak, ai, bk, bi)
        if dir_all:
            ta = aw
        else:
            dv = jnp.full((L,), True) == jax.lax.broadcast((p & k) == 0, (L,))
            ta = aw == dv
        key_v[pl.ds(p, L)] = jnp.where(ta, ak, bk)
        idx_v[pl.ds(p, L)] = jnp.where(ta, ai, bi)
        key_v[pl.ds(p + j, L)] = jnp.where(ta, bk, ak)
        idx_v[pl.ds(p + j, L)] = jnp.where(ta, bi, ai)


def _intra_pass(key_v, idx_v, k, nvec, dir_all, start=0):
    iota = lax.iota(jnp.int32, L)

    @plsc.parallel_loop(0, nvec, unroll=2)
    def _(v):
        base = start + v * L
        mk = key_v[pl.ds(base, L)]
        mi = idx_v[pl.ds(base, L)]
        if dir_all:
            dm = iota >= 0
        else:
            dm = ((base + iota) & k) == 0
        for j in (8, 4, 2, 1):
            perm = iota ^ j
            pk = _lane_shuffle(mk, perm)
            pi = _lane_shuffle(mi, perm)
            g = _tot_gt(mk, mi, pk, pi)
            is_low = (iota & j) == 0
            keep_mine = (dm == is_low) == g
            mk = jnp.where(keep_mine, mk, pk)
            mi = jnp.where(keep_mine, mi, pi)
        key_v[pl.ds(base, L)] = mk
        idx_v[pl.ds(base, L)] = mi


def _vreg_sort16(key_v, idx_v):
    iota = lax.iota(jnp.int32, L)
    stages = [(2, 1), (4, 2), (4, 1), (8, 4), (8, 2), (8, 1),
              (16, 8), (16, 4), (16, 2), (16, 1)]

    @plsc.parallel_loop(0, NV, unroll=2)
    def _(v):
        base = v * L
        mk = key_v[pl.ds(base, L)]
        mi = idx_v[pl.ds(base, L)]
        for k, j in stages:
            dm = ((base + iota) & k) == 0
            perm = iota ^ j
            pk = _lane_shuffle(mk, perm)
            pi = _lane_shuffle(mi, perm)
            g = _tot_gt(mk, mi, pk, pi)
            is_low = (iota & j) == 0
            keep_mine = (dm == is_low) == g
            mk = jnp.where(keep_mine, mk, pk)
            mi = jnp.where(keep_mine, mi, pi)
        key_v[pl.ds(base, L)] = mk
        idx_v[pl.ds(base, L)] = mi


def _local_sort(key_v, idx_v):
    _vreg_sort16(key_v, idx_v)

    def level(kk, carry):
        k = 1 << kk

        def cross(i, carry2):
            _cross_stage(key_v, idx_v, k, k >> (i + 1), CHUNK, dir_all=False)
            return carry2

        lax.fori_loop(0, kk - 4, cross, 0)
        _intra_pass(key_v, idx_v, k, NV, dir_all=False)
        return carry

    lax.fori_loop(5, 11, level, 0)


def _merge_prune(key_v, idx_v, pk_v, pi_v, h):
    iota = lax.iota(jnp.int32, L)

    @plsc.parallel_loop(0, NV, unroll=2)
    def _(v):
        base = v * L
        ak = key_v[pl.ds(base, L)]
        ai = idx_v[pl.ds(base, L)]
        rev = (CHUNK - 1 - base) - iota
        bk = plsc.load_gather(pk_v, [rev])
        bi = plsc.load_gather(pi_v, [rev])
        aw = _tot_gt(ak, ai, bk, bi)
        key_v[pl.ds(base, L)] = jnp.where(aw, ak, bk)
        idx_v[pl.ds(base, L)] = jnp.where(aw, ai, bi)

    _cross_stage(key_v, idx_v, 0, 512, CHUNK, dir_all=True)
    half = h * (CHUNK // 2)

    def cross(i, carry):
        _cross_stage(key_v, idx_v, 0, 256 >> i, CHUNK // 2, dir_all=True,
                     start=half)
        return carry

    lax.fori_loop(0, 5, cross, 0)
    _intra_pass(key_v, idx_v, 0, NV // 2, dir_all=True, start=half)


def _sc_body(sco_hbm, x_hbm, xv_hbm, xg_hbm, xvg_hbm, sg_hbm,
             key_v, idx_v, pk_v, pi_v, idxg_v, rows_v,
             keys_sh, idx_sh, sem):
    c = lax.axis_index("c")
    s = lax.axis_index("s")
    g = s // 8
    q = s % 8
    b = c * 2 + g
    iota = lax.iota(jnp.int32, L)

    base_in = b * N + q * CHUNK
    pltpu.sync_copy(sco_hbm.at[pl.ds(base_in, CHUNK)], key_v)

    @plsc.parallel_loop(0, NV, unroll=2)
    def _(v):
        idx_v[pl.ds(v * L, L)] = (q * CHUNK + v * L) + iota

    _local_sort(key_v, idx_v)

    pltpu.sync_copy(key_v, keys_sh.at[s])
    pltpu.sync_copy(idx_v, idx_sh.at[s])
    plsc.subcore_barrier()

    for r in range(3):
        nmerge = 4 >> r
        active = q < 2 * nmerge
        m = q % nmerge
        h = q // nmerge
        half = CHUNK // 2

        @pl.when(active)
        def _read():
            slot = g * 8 + 2 * m
            pltpu.sync_copy(keys_sh.at[slot], key_v)
            pltpu.sync_copy(idx_sh.at[slot], idx_v)
            pltpu.sync_copy(keys_sh.at[slot + 1], pk_v)
            pltpu.sync_copy(idx_sh.at[slot + 1], pi_v)

        plsc.subcore_barrier()

        @pl.when(active)
        def _merge():
            _merge_prune(key_v, idx_v, pk_v, pi_v, h)
            hs = h * half
            pltpu.sync_copy(key_v.at[pl.ds(hs, half)],
                            keys_sh.at[g * 8 + m, pl.ds(hs, half)])
            pltpu.sync_copy(idx_v.at[pl.ds(hs, half)],
                            idx_sh.at[g * 8 + m, pl.ds(hs, half)])

        plsc.subcore_barrier()

    @pl.when(q == 0)
    def _write_scores():
        pltpu.sync_copy(keys_sh.at[g * 8], sg_hbm.at[pl.ds(b * K, K)])

    rows_per_tile = K // 8
    pltpu.sync_copy(idx_sh.at[g * 8, pl.ds(q * rows_per_tile, rows_per_tile)],
                    idxg_v)
    for t in range(rows_per_tile // L):
        idxg_v[pl.ds(t * L, L)] = idxg_v[pl.ds(t * L, L)] + b * N
    out_base = b * K + q * rows_per_tile
    pltpu.async_copy(x_hbm.at[idxg_v], rows_v, sem).wait()
    pltpu.sync_copy(rows_v, xg_hbm.at[pl.ds(out_base, rows_per_tile)])
    pltpu.async_copy(xv_hbm.at[idxg_v], rows_v, sem).wait()
    pltpu.sync_copy(rows_v, xvg_hbm.at[pl.ds(out_base, rows_per_tile)])


_sc_topk = functools.partial(
    pl.kernel,
    out_type=(
        jax.ShapeDtypeStruct((B * K, D), jnp.float32),
        jax.ShapeDtypeStruct((B * K, D), jnp.float32),
        jax.ShapeDtypeStruct((B * K,), jnp.int32),
    ),
    mesh=plsc.VectorSubcoreMesh(core_axis_name="c", subcore_axis_name="s"),
    compiler_params=pltpu.CompilerParams(needs_layout_passes=False),
    scratch_types=[
        pltpu.VMEM((CHUNK,), jnp.int32),
        pltpu.VMEM((CHUNK,), jnp.int32),
        pltpu.VMEM((CHUNK,), jnp.int32),
        pltpu.VMEM((CHUNK,), jnp.int32),
        pltpu.VMEM((K // 8,), jnp.int32),
        pltpu.VMEM((K // 8, D), jnp.float32),
        pltpu.VMEM_SHARED((NS, CHUNK), jnp.int32),
        pltpu.VMEM_SHARED((NS, CHUNK), jnp.int32),
        pltpu.SemaphoreType.DMA,
    ],
)(_sc_body)


@jax.jit
def kernel(x, x_v, W1_w, W1_b, V_w, V_b):
    s = _scores(x, W1_w, W1_b, V_w, V_b)
    xg, xvg, sg = _sc_topk(s.reshape(B * N),
                           x.reshape(B * N, D),
                           x_v.reshape(B * N, D))
    sgf = lax.bitcast_convert_type(sg, jnp.float32)
    return (xg.reshape(B, K, D), xvg.reshape(B, K, D),
            sgf.reshape(B, K, 1))


# speedup vs baseline: 1.7164x; 1.0764x over previous
import functools
import jax
import jax.numpy as jnp
from jax import lax
from jax.experimental import pallas as pl
from jax.experimental.pallas import tpu as pltpu
from jax.experimental.pallas import tpu_sc as plsc

POOL = 0.125
NC, NS, L = 2, 16, 16
B, N, D = 4, 8192, 128
K = int(N * POOL)
CHUNK = N // 8
NV = CHUNK // L


def _score_body(x_ref, w1_ref, b1_ref, v_ref, vb_ref, s_ref):
    Bb, BN, Dd = x_ref.shape
    H = w1_ref.shape[0]
    x = x_ref[...].reshape(Bb * BN, Dd)
    h = jnp.tanh(lax.dot_general(
        x, w1_ref[...], (((1,), (1,)), ((), ())),
        preferred_element_type=jnp.float32) + b1_ref[...][None, :])
    row = lax.broadcasted_iota(jnp.int32, (128, H), 0)
    v_pad = jnp.where(row == 0, v_ref[...][0][None, :] + 0.0 * row, 0.0)
    logit = lax.dot_general(
        h, v_pad, (((1,), (1,)), ((), ())),
        preferred_element_type=jnp.float32) + vb_ref[0]
    sig = jax.nn.sigmoid(logit[:, 0]).reshape(Bb, BN)
    s_ref[...] = lax.bitcast_convert_type(sig, jnp.int32)


def _scores(x, W1_w, W1_b, V_w, V_b):
    H = W1_w.shape[0]
    BN = 2048
    return pl.pallas_call(
        _score_body,
        grid=(N // BN,),
        in_specs=[
            pl.BlockSpec((B, BN, D), lambda n: (0, n, 0)),
            pl.BlockSpec((H, D), lambda n: (0, 0)),
            pl.BlockSpec((H,), lambda n: (0,)),
            pl.BlockSpec((1, H), lambda n: (0, 0)),
            pl.BlockSpec((1,), lambda n: (0,)),
        ],
        out_specs=pl.BlockSpec((B, BN), lambda n: (0, n)),
        out_shape=jax.ShapeDtypeStruct((B, N), jnp.int32),
    )(x, W1_w, W1_b, V_w, V_b)


_GDN = lax.GatherDimensionNumbers(
    offset_dims=(), collapsed_slice_dims=(0,), start_index_map=(0,))


def _lane_shuffle(vec, perm):
    return lax.gather(vec, perm[:, None], _GDN, (1,),
                      mode=lax.GatherScatterMode.PROMISE_IN_BOUNDS)


def _tot_gt(ak, ai, bk, bi):
    return jnp.logical_or(ak > bk, jnp.logical_and(ak == bk, ai < bi))


def _cross_stage(key_v, idx_v, k, j, npos, dir_all, start=0):
    npairs = npos // 2 // L

    @plsc.parallel_loop(0, npairs, unroll=2)
    def _(t):
        t16 = t * L
        jm1 = j - 1
        p = start + (((t16 & ~jm1) << 1) | (t16 & jm1))
        ak = key_v[pl.ds(p, L)]
        ai = idx_v[pl.ds(p, L)]
        bk = key_v[pl.ds(p + j, L)]
        bi = idx_v[pl.ds(p + j, L)]
        aw = _tot_gt(ak, ai, bk, bi)
        if dir_all:
            ta = aw
        else:
            dv = jnp.full((L,), True) == jax.lax.broadcast((p & k) == 0, (L,))
            ta = aw == dv
        key_v[pl.ds(p, L)] = jnp.where(ta, ak, bk)
        idx_v[pl.ds(p, L)] = jnp.where(ta, ai, bi)
        key_v[pl.ds(p + j, L)] = jnp.where(ta, bk, ak)
        idx_v[pl.ds(p + j, L)] = jnp.where(ta, bi, ai)


def _intra_pass(key_v, idx_v, k, nvec, dir_all, start=0):
    iota = lax.iota(jnp.int32, L)

    @plsc.parallel_loop(0, nvec, unroll=2)
    def _(v):
        base = start + v * L
        mk = key_v[pl.ds(base, L)]
        mi = idx_v[pl.ds(base, L)]
        if dir_all:
            dm = iota >= 0
        else:
            dm = ((base + iota) & k) == 0
        for j in (8, 4, 2, 1):
            perm = iota ^ j
            pk = _lane_shuffle(mk, perm)
            pi = _lane_shuffle(mi, perm)
            g = _tot_gt(mk, mi, pk, pi)
            is_low = (iota & j) == 0
            keep_mine = (dm == is_low) == g
            mk = jnp.where(keep_mine, mk, pk)
            mi = jnp.where(keep_mine, mi, pi)
        key_v[pl.ds(base, L)] = mk
        idx_v[pl.ds(base, L)] = mi


def _vreg_sort16(key_v, idx_v):
    iota = lax.iota(jnp.int32, L)
    stages = [(2, 1), (4, 2), (4, 1), (8, 4), (8, 2), (8, 1),
              (16, 8), (16, 4), (16, 2), (16, 1)]

    @plsc.parallel_loop(0, NV, unroll=2)
    def _(v):
        base = v * L
        mk = key_v[pl.ds(base, L)]
        mi = idx_v[pl.ds(base, L)]
        for k, j in stages:
            dm = ((base + iota) & k) == 0
            perm = iota ^ j
            pk = _lane_shuffle(mk, perm)
            pi = _lane_shuffle(mi, perm)
            g = _tot_gt(mk, mi, pk, pi)
            is_low = (iota & j) == 0
            keep_mine = (dm == is_low) == g
            mk = jnp.where(keep_mine, mk, pk)
            mi = jnp.where(keep_mine, mi, pi)
        key_v[pl.ds(base, L)] = mk
        idx_v[pl.ds(base, L)] = mi


def _local_sort(key_v, idx_v):
    _vreg_sort16(key_v, idx_v)

    def level(kk, carry):
        k = 1 << kk

        def cross(i, carry2):
            _cross_stage(key_v, idx_v, k, k >> (i + 1), CHUNK, dir_all=False)
            return carry2

        lax.fori_loop(0, kk - 4, cross, 0)
        _intra_pass(key_v, idx_v, k, NV, dir_all=False)
        return carry

    lax.fori_loop(5, 11, level, 0)


def _merge_prune(key_v, idx_v, pk_v, pi_v, h):
    iota = lax.iota(jnp.int32, L)

    @plsc.parallel_loop(0, NV, unroll=2)
    def _(v):
        base = v * L
        ak = key_v[pl.ds(base, L)]
        ai = idx_v[pl.ds(base, L)]
        rev = (CHUNK - 1 - base) - iota
        bk = plsc.load_gather(pk_v, [rev])
        bi = plsc.load_gather(pi_v, [rev])
        aw = _tot_gt(ak, ai, bk, bi)
        key_v[pl.ds(base, L)] = jnp.where(aw, ak, bk)
        idx_v[pl.ds(base, L)] = jnp.where(aw, ai, bi)

    _cross_stage(key_v, idx_v, 0, 512, CHUNK, dir_all=True)
    half = h * (CHUNK // 2)

    def cross(i, carry):
        _cross_stage(key_v, idx_v, 0, 256 >> i, CHUNK // 2, dir_all=True,
                     start=half)
        return carry

    lax.fori_loop(0, 5, cross, 0)
    _intra_pass(key_v, idx_v, 0, NV // 2, dir_all=True, start=half)


def _sc_body(sco_hbm, x_hbm, xv_hbm, xg_hbm, xvg_hbm, sg_hbm,
             key_v, idx_v, pk_v, pi_v, idxg_v, rows_v, rows2_v, sco_f,
             keys_sh, idx_sh, sem, sem2, sem3, sem4):
    c = lax.axis_index("c")
    s = lax.axis_index("s")
    g = s // 8
    q = s % 8
    b = c * 2 + g
    iota = lax.iota(jnp.int32, L)

    base_in = b * N + q * CHUNK
    pltpu.sync_copy(sco_hbm.at[pl.ds(base_in, CHUNK)], key_v)

    @plsc.parallel_loop(0, NV, unroll=2)
    def _(v):
        idx_v[pl.ds(v * L, L)] = (q * CHUNK + v * L) + iota

    _local_sort(key_v, idx_v)

    pltpu.sync_copy(key_v, keys_sh.at[s])
    pltpu.sync_copy(idx_v, idx_sh.at[s])
    plsc.subcore_barrier()

    for r in range(3):
        nmerge = 4 >> r
        active = q < 2 * nmerge
        m = q % nmerge
        h = q // nmerge
        half = CHUNK // 2

        @pl.when(active)
        def _read():
            slot = g * 8 + 2 * m
            pltpu.sync_copy(keys_sh.at[slot], key_v)
            pltpu.sync_copy(idx_sh.at[slot], idx_v)
            pltpu.sync_copy(keys_sh.at[slot + 1], pk_v)
            pltpu.sync_copy(idx_sh.at[slot + 1], pi_v)

        plsc.subcore_barrier()

        @pl.when(active)
        def _merge():
            _merge_prune(key_v, idx_v, pk_v, pi_v, h)
            hs = h * half
            pltpu.sync_copy(key_v.at[pl.ds(hs, half)],
                            keys_sh.at[g * 8 + m, pl.ds(hs, half)])
            pltpu.sync_copy(idx_v.at[pl.ds(hs, half)],
                            idx_sh.at[g * 8 + m, pl.ds(hs, half)])

        plsc.subcore_barrier()

    half = CHUNK // 2

    @pl.when(q < 2)
    def _write_scores():
        hs = q * half

        @plsc.parallel_loop(0, NV // 2, unroll=2)
        def _(v):
            sco_f[pl.ds(v * L, L)] = plsc.bitcast(
                key_v[pl.ds(hs + v * L, L)], jnp.float32)

        pltpu.sync_copy(sco_f, sg_hbm.at[pl.ds(b * K + hs, half)])

    rows_per_tile = K // 8
    pltpu.sync_copy(idx_sh.at[g * 8, pl.ds(q * rows_per_tile, rows_per_tile)],
                    idxg_v)
    for t in range(rows_per_tile // L):
        idxg_v[pl.ds(t * L, L)] = idxg_v[pl.ds(t * L, L)] + b * N
    out_base = b * K + q * rows_per_tile
    g1 = pltpu.async_copy(x_hbm.at[idxg_v], rows_v, sem)
    g2 = pltpu.async_copy(xv_hbm.at[idxg_v], rows2_v, sem2)
    g1.wait()
    w1 = pltpu.async_copy(rows_v, xg_hbm.at[pl.ds(out_base, rows_per_tile)],
                          sem3)
    g2.wait()
    w2 = pltpu.async_copy(rows2_v,
                          xvg_hbm.at[pl.ds(out_base, rows_per_tile)], sem4)
    w1.wait()
    w2.wait()


_sc_topk = functools.partial(
    pl.kernel,
    out_type=(
        jax.ShapeDtypeStruct((B * K, D), jnp.float32),
        jax.ShapeDtypeStruct((B * K, D), jnp.float32),
        jax.ShapeDtypeStruct((B * K,), jnp.float32),
    ),
    mesh=plsc.VectorSubcoreMesh(core_axis_name="c", subcore_axis_name="s"),
    compiler_params=pltpu.CompilerParams(needs_layout_passes=False),
    scratch_types=[
        pltpu.VMEM((CHUNK,), jnp.int32),
        pltpu.VMEM((CHUNK,), jnp.int32),
        pltpu.VMEM((CHUNK,), jnp.int32),
        pltpu.VMEM((CHUNK,), jnp.int32),
        pltpu.VMEM((K // 8,), jnp.int32),
        pltpu.VMEM((K // 8, D), jnp.float32),
        pltpu.VMEM((K // 8, D), jnp.float32),
        pltpu.VMEM((CHUNK // 2,), jnp.float32),
        pltpu.VMEM_SHARED((NS, CHUNK), jnp.int32),
        pltpu.VMEM_SHARED((NS, CHUNK), jnp.int32),
        pltpu.SemaphoreType.DMA,
        pltpu.SemaphoreType.DMA,
        pltpu.SemaphoreType.DMA,
        pltpu.SemaphoreType.DMA,
    ],
)(_sc_body)


@jax.jit
def kernel(x, x_v, W1_w, W1_b, V_w, V_b):
    s = _scores(x, W1_w, W1_b, V_w, V_b)
    xg, xvg, sg = _sc_topk(s.reshape(B * N),
                           x.reshape(B * N, D),
                           x_v.reshape(B * N, D))
    return (xg.reshape(B, K, D), xvg.reshape(B, K, D),
            sg.reshape(B, K, 1))


# speedup vs baseline: 1.7385x; 1.0129x over previous
import functools
import jax
import jax.numpy as jnp
from jax import lax
from jax.experimental import pallas as pl
from jax.experimental.pallas import tpu as pltpu
from jax.experimental.pallas import tpu_sc as plsc

POOL = 0.125
NC, NS, L = 2, 16, 16
B, N, D = 4, 8192, 128
K = int(N * POOL)
CHUNK = N // 8
NV = CHUNK // L


def _score_body(x_ref, w1_ref, b1_ref, v_ref, vb_ref, s_ref):
    Bb, BN, Dd = x_ref.shape
    H = w1_ref.shape[0]
    x = x_ref[...].reshape(Bb * BN, Dd)
    h = jnp.tanh(lax.dot_general(
        x, w1_ref[...], (((1,), (1,)), ((), ())),
        preferred_element_type=jnp.float32) + b1_ref[...][None, :])
    row = lax.broadcasted_iota(jnp.int32, (128, H), 0)
    v_pad = jnp.where(row == 0, v_ref[...][0][None, :] + 0.0 * row, 0.0)
    logit = lax.dot_general(
        h, v_pad, (((1,), (1,)), ((), ())),
        preferred_element_type=jnp.float32) + vb_ref[0]
    sig = jax.nn.sigmoid(logit[:, 0]).reshape(Bb, BN)
    s_ref[...] = lax.bitcast_convert_type(sig, jnp.int32)


def _scores(x, W1_w, W1_b, V_w, V_b):
    H = W1_w.shape[0]
    BN = 2048
    return pl.pallas_call(
        _score_body,
        grid=(N // BN,),
        in_specs=[
            pl.BlockSpec((B, BN, D), lambda n: (0, n, 0)),
            pl.BlockSpec((H, D), lambda n: (0, 0)),
            pl.BlockSpec((H,), lambda n: (0,)),
            pl.BlockSpec((1, H), lambda n: (0, 0)),
            pl.BlockSpec((1,), lambda n: (0,)),
        ],
        out_specs=pl.BlockSpec((B, BN), lambda n: (0, n)),
        out_shape=jax.ShapeDtypeStruct((B, N), jnp.int32),
    )(x, W1_w, W1_b, V_w, V_b)


_GDN = lax.GatherDimensionNumbers(
    offset_dims=(), collapsed_slice_dims=(0,), start_index_map=(0,))


def _lane_shuffle(vec, perm):
    return lax.gather(vec, perm[:, None], _GDN, (1,),
                      mode=lax.GatherScatterMode.PROMISE_IN_BOUNDS)


def _tot_gt(ak, ai, bk, bi):
    return jnp.logical_or(ak > bk, jnp.logical_and(ak == bk, ai < bi))


def _cross_stage(key_v, idx_v, k, j, npos, dir_all, start=0):
    npairs = npos // 2 // L

    @plsc.parallel_loop(0, npairs, unroll=2)
    def _(t):
        t16 = t * L
        jm1 = j - 1
        p = start + (((t16 & ~jm1) << 1) | (t16 & jm1))
        ak = key_v[pl.ds(p, L)]
        ai = idx_v[pl.ds(p, L)]
        bk = key_v[pl.ds(p + j, L)]
        bi = idx_v[pl.ds(p + j, L)]
        aw = _tot_gt(ak, ai, bk, bi)
        if dir_all:
            ta = aw
        else:
            dv = jnp.full((L,), True) == jax.lax.broadcast((p & k) == 0, (L,))
            ta = aw == dv
        key_v[pl.ds(p, L)] = jnp.where(ta, ak, bk)
        idx_v[pl.ds(p, L)] = jnp.where(ta, ai, bi)
        key_v[pl.ds(p + j, L)] = jnp.where(ta, bk, ak)
        idx_v[pl.ds(p + j, L)] = jnp.where(ta, bi, ai)


def _intra_pass(key_v, idx_v, k, nvec, dir_all, start=0):
    iota = lax.iota(jnp.int32, L)

    @plsc.parallel_loop(0, nvec, unroll=2)
    def _(v):
        base = start + v * L
        mk = key_v[pl.ds(base, L)]
        mi = idx_v[pl.ds(base, L)]
        if dir_all:
            dm = iota >= 0
        else:
            dm = ((base + iota) & k) == 0
        for j in (8, 4, 2, 1):
            perm = iota ^ j
            pk = _lane_shuffle(mk, perm)
            pi = _lane_shuffle(mi, perm)
            g = _tot_gt(mk, mi, pk, pi)
            is_low = (iota & j) == 0
            keep_mine = (dm == is_low) == g
            mk = jnp.where(keep_mine, mk, pk)
            mi = jnp.where(keep_mine, mi, pi)
        key_v[pl.ds(base, L)] = mk
        idx_v[pl.ds(base, L)] = mi


def _vreg_sort16(key_v, idx_v):
    iota = lax.iota(jnp.int32, L)
    stages = [(2, 1), (4, 2), (4, 1), (8, 4), (8, 2), (8, 1),
              (16, 8), (16, 4), (16, 2), (16, 1)]

    @plsc.parallel_loop(0, NV, unroll=2)
    def _(v):
        base = v * L
        mk = key_v[pl.ds(base, L)]
        mi = idx_v[pl.ds(base, L)]
        for k, j in stages:
            dm = ((base + iota) & k) == 0
            perm = iota ^ j
            pk = _lane_shuffle(mk, perm)
            pi = _lane_shuffle(mi, perm)
            g = _tot_gt(mk, mi, pk, pi)
            is_low = (iota & j) == 0
            keep_mine = (dm == is_low) == g
            mk = jnp.where(keep_mine, mk, pk)
            mi = jnp.where(keep_mine, mi, pi)
        key_v[pl.ds(base, L)] = mk
        idx_v[pl.ds(base, L)] = mi


def _local_sort(key_v, idx_v):
    _vreg_sort16(key_v, idx_v)

    def level(kk, carry):
        k = 1 << kk

        def cross(i, carry2):
            _cross_stage(key_v, idx_v, k, k >> (i + 1), CHUNK, dir_all=False)
            return carry2

        lax.fori_loop(0, kk - 4, cross, 0)
        _intra_pass(key_v, idx_v, k, NV, dir_all=False)
        return carry

    lax.fori_loop(5, 11, level, 0)


def _merge_prune(key_v, idx_v, pk_v, pi_v, h):
    iota = lax.iota(jnp.int32, L)

    @plsc.parallel_loop(0, NV, unroll=2)
    def _(v):
        base = v * L
        ak = key_v[pl.ds(base, L)]
        ai = idx_v[pl.ds(base, L)]
        rev = (CHUNK - 1 - base) - iota
        bk = plsc.load_gather(pk_v, [rev])
        bi = plsc.load_gather(pi_v, [rev])
        aw = _tot_gt(ak, ai, bk, bi)
        key_v[pl.ds(base, L)] = jnp.where(aw, ak, bk)
        idx_v[pl.ds(base, L)] = jnp.where(aw, ai, bi)

    _cross_stage(key_v, idx_v, 0, 512, CHUNK, dir_all=True)
    half = h * (CHUNK // 2)

    def cross(i, carry):
        _cross_stage(key_v, idx_v, 0, 256 >> i, CHUNK // 2, dir_all=True,
                     start=half)
        return carry

    lax.fori_loop(0, 5, cross, 0)
    _intra_pass(key_v, idx_v, 0, NV // 2, dir_all=True, start=half)


def _sc_body(sco_hbm, x_hbm, xv_hbm, xg_hbm, xvg_hbm, sg_hbm,
             key_v, idx_v, pk_v, pi_v, idxg_v, rows_v, rows2_v, sco_f,
             keys_sh, idx_sh, sem, sem2, sem3, sem4):
    c = lax.axis_index("c")
    s = lax.axis_index("s")
    g = s // 8
    q = s % 8
    b = c * 2 + g
    iota = lax.iota(jnp.int32, L)

    base_in = b * N + q * CHUNK
    pltpu.sync_copy(sco_hbm.at[pl.ds(base_in, CHUNK)], key_v)

    @plsc.parallel_loop(0, NV, unroll=2)
    def _(v):
        idx_v[pl.ds(v * L, L)] = (q * CHUNK + v * L) + iota

    _local_sort(key_v, idx_v)

    pltpu.sync_copy(key_v, keys_sh.at[s])
    pltpu.sync_copy(idx_v, idx_sh.at[s])
    plsc.subcore_barrier()

    half = CHUNK // 2

    def _round(r, carry):
        nmerge = 4 >> r
        active = q < 2 * nmerge
        m = q & (nmerge - 1)
        h = q >> (2 - r)

        @pl.when(active)
        def _read():
            slot = g * 8 + 2 * m
            pltpu.sync_copy(keys_sh.at[slot], key_v)
            pltpu.sync_copy(idx_sh.at[slot], idx_v)
            pltpu.sync_copy(keys_sh.at[slot + 1], pk_v)
            pltpu.sync_copy(idx_sh.at[slot + 1], pi_v)

        plsc.subcore_barrier()

        @pl.when(active)
        def _merge():
            _merge_prune(key_v, idx_v, pk_v, pi_v, h)
            hs = h * half
            pltpu.sync_copy(key_v.at[pl.ds(hs, half)],
                            keys_sh.at[g * 8 + m, pl.ds(hs, half)])
            pltpu.sync_copy(idx_v.at[pl.ds(hs, half)],
                            idx_sh.at[g * 8 + m, pl.ds(hs, half)])

        plsc.subcore_barrier()
        return carry

    lax.fori_loop(0, 3, _round, 0)

    half = CHUNK // 2

    @pl.when(q < 2)
    def _write_scores():
        hs = q * half

        @plsc.parallel_loop(0, NV // 2, unroll=2)
        def _(v):
            sco_f[pl.ds(v * L, L)] = plsc.bitcast(
                key_v[pl.ds(hs + v * L, L)], jnp.float32)

        pltpu.sync_copy(sco_f, sg_hbm.at[pl.ds(b * K + hs, half)])

    rows_per_tile = K // 8
    pltpu.sync_copy(idx_sh.at[g * 8, pl.ds(q * rows_per_tile, rows_per_tile)],
                    idxg_v)
    for t in range(rows_per_tile // L):
        idxg_v[pl.ds(t * L, L)] = idxg_v[pl.ds(t * L, L)] + b * N
    out_base = b * K + q * rows_per_tile
    g1 = pltpu.async_copy(x_hbm.at[idxg_v], rows_v, sem)
    g2 = pltpu.async_copy(xv_hbm.at[idxg_v], rows2_v, sem2)
    g1.wait()
    w1 = pltpu.async_copy(rows_v, xg_hbm.at[pl.ds(out_base, rows_per_tile)],
                          sem3)
    g2.wait()
    w2 = pltpu.async_copy(rows2_v,
                          xvg_hbm.at[pl.ds(out_base, rows_per_tile)], sem4)
    w1.wait()
    w2.wait()


_sc_topk = functools.partial(
    pl.kernel,
    out_type=(
        jax.ShapeDtypeStruct((B * K, D), jnp.float32),
        jax.ShapeDtypeStruct((B * K, D), jnp.float32),
        jax.ShapeDtypeStruct((B * K,), jnp.float32),
    ),
    mesh=plsc.VectorSubcoreMesh(core_axis_name="c", subcore_axis_name="s"),
    compiler_params=pltpu.CompilerParams(needs_layout_passes=False),
    scratch_types=[
        pltpu.VMEM((CHUNK,), jnp.int32),
        pltpu.VMEM((CHUNK,), jnp.int32),
        pltpu.VMEM((CHUNK,), jnp.int32),
        pltpu.VMEM((CHUNK,), jnp.int32),
        pltpu.VMEM((K // 8,), jnp.int32),
        pltpu.VMEM((K // 8, D), jnp.float32),
        pltpu.VMEM((K // 8, D), jnp.float32),
        pltpu.VMEM((CHUNK // 2,), jnp.float32),
        pltpu.VMEM_SHARED((NS, CHUNK), jnp.int32),
        pltpu.VMEM_SHARED((NS, CHUNK), jnp.int32),
        pltpu.SemaphoreType.DMA,
        pltpu.SemaphoreType.DMA,
        pltpu.SemaphoreType.DMA,
        pltpu.SemaphoreType.DMA,
    ],
)(_sc_body)


@jax.jit
def kernel(x, x_v, W1_w, W1_b, V_w, V_b):
    s = _scores(x, W1_w, W1_b, V_w, V_b)
    xg, xvg, sg = _sc_topk(s.reshape(B * N),
                           x.reshape(B * N, D),
                           x_v.reshape(B * N, D))
    return (xg.reshape(B, K, D), xvg.reshape(B, K, D),
            sg.reshape(B, K, 1))


# speedup vs baseline: 1.7436x; 1.0029x over previous
import functools
import jax
import jax.numpy as jnp
from jax import lax
from jax.experimental import pallas as pl
from jax.experimental.pallas import tpu as pltpu
from jax.experimental.pallas import tpu_sc as plsc

POOL = 0.125
NC, NS, L = 2, 16, 16
B, N, D = 4, 8192, 128
K = int(N * POOL)
CHUNK = N // 8
NV = CHUNK // L


def _score_body(x_ref, w1_ref, b1_ref, v_ref, vb_ref, s_ref):
    Bb, BN, Dd = x_ref.shape
    H = w1_ref.shape[0]
    x = x_ref[...].reshape(Bb * BN, Dd)
    h = jnp.tanh(lax.dot_general(
        x, w1_ref[...], (((1,), (1,)), ((), ())),
        preferred_element_type=jnp.float32) + b1_ref[...][None, :])
    row = lax.broadcasted_iota(jnp.int32, (128, H), 0)
    v_pad = jnp.where(row == 0, v_ref[...][0][None, :] + 0.0 * row, 0.0)
    logit = lax.dot_general(
        h, v_pad, (((1,), (1,)), ((), ())),
        preferred_element_type=jnp.float32) + vb_ref[0]
    sig = jax.nn.sigmoid(logit[:, 0]).reshape(Bb, BN)
    s_ref[...] = lax.bitcast_convert_type(sig, jnp.int32)


def _scores(x, W1_w, W1_b, V_w, V_b):
    H = W1_w.shape[0]
    BN = 2048
    return pl.pallas_call(
        _score_body,
        grid=(N // BN,),
        in_specs=[
            pl.BlockSpec((B, BN, D), lambda n: (0, n, 0)),
            pl.BlockSpec((H, D), lambda n: (0, 0)),
            pl.BlockSpec((H,), lambda n: (0,)),
            pl.BlockSpec((1, H), lambda n: (0, 0)),
            pl.BlockSpec((1,), lambda n: (0,)),
        ],
        out_specs=pl.BlockSpec((B, BN), lambda n: (0, n)),
        out_shape=jax.ShapeDtypeStruct((B, N), jnp.int32),
    )(x, W1_w, W1_b, V_w, V_b)


_GDN = lax.GatherDimensionNumbers(
    offset_dims=(), collapsed_slice_dims=(0,), start_index_map=(0,))


def _lane_shuffle(vec, perm):
    return lax.gather(vec, perm[:, None], _GDN, (1,),
                      mode=lax.GatherScatterMode.PROMISE_IN_BOUNDS)


def _tot_gt(ak, ai, bk, bi):
    return jnp.logical_or(ak > bk, jnp.logical_and(ak == bk, ai < bi))


def _cross_stage(key_v, idx_v, k, j, npos, dir_all, start=0):
    npairs = npos // 2 // L

    @plsc.parallel_loop(0, npairs, unroll=2)
    def _(t):
        t16 = t * L
        jm1 = j - 1
        p = start + (((t16 & ~jm1) << 1) | (t16 & jm1))
        ak = key_v[pl.ds(p, L)]
        ai = idx_v[pl.ds(p, L)]
        bk = key_v[pl.ds(p + j, L)]
        bi = idx_v[pl.ds(p + j, L)]
        aw = _tot_gt(ak, ai, bk, bi)
        if dir_all:
            ta = aw
        else:
            dv = jnp.full((L,), True) == jax.lax.broadcast((p & k) == 0, (L,))
            ta = aw == dv
        key_v[pl.ds(p, L)] = jnp.where(ta, ak, bk)
        idx_v[pl.ds(p, L)] = jnp.where(ta, ai, bi)
        key_v[pl.ds(p + j, L)] = jnp.where(ta, bk, ak)
        idx_v[pl.ds(p + j, L)] = jnp.where(ta, bi, ai)


def _intra_pass(key_v, idx_v, k, nvec, dir_all, start=0):
    iota = lax.iota(jnp.int32, L)

    @plsc.parallel_loop(0, nvec, unroll=4)
    def _(v):
        base = start + v * L
        mk = key_v[pl.ds(base, L)]
        mi = idx_v[pl.ds(base, L)]
        if dir_all:
            dm = iota >= 0
        else:
            dm = ((base + iota) & k) == 0
        for j in (8, 4, 2, 1):
            perm = iota ^ j
            pk = _lane_shuffle(mk, perm)
            pi = _lane_shuffle(mi, perm)
            g = _tot_gt(mk, mi, pk, pi)
            is_low = (iota & j) == 0
            keep_mine = (dm == is_low) == g
            mk = jnp.where(keep_mine, mk, pk)
            mi = jnp.where(keep_mine, mi, pi)
        key_v[pl.ds(base, L)] = mk
        idx_v[pl.ds(base, L)] = mi


def _vreg_sort16(key_v, idx_v):
    iota = lax.iota(jnp.int32, L)
    stages = [(2, 1), (4, 2), (4, 1), (8, 4), (8, 2), (8, 1),
              (16, 8), (16, 4), (16, 2), (16, 1)]

    @plsc.parallel_loop(0, NV, unroll=4)
    def _(v):
        base = v * L
        mk = key_v[pl.ds(base, L)]
        mi = idx_v[pl.ds(base, L)]
        for k, j in stages:
            dm = ((base + iota) & k) == 0
            perm = iota ^ j
            pk = _lane_shuffle(mk, perm)
            pi = _lane_shuffle(mi, perm)
            g = _tot_gt(mk, mi, pk, pi)
            is_low = (iota & j) == 0
            keep_mine = (dm == is_low) == g
            mk = jnp.where(keep_mine, mk, pk)
            mi = jnp.where(keep_mine, mi, pi)
        key_v[pl.ds(base, L)] = mk
        idx_v[pl.ds(base, L)] = mi


def _local_sort(key_v, idx_v):
    _vreg_sort16(key_v, idx_v)

    def level(kk, carry):
        k = 1 << kk

        def cross(i, carry2):
            _cross_stage(key_v, idx_v, k, k >> (i + 1), CHUNK, dir_all=False)
            return carry2

        lax.fori_loop(0, kk - 4, cross, 0)
        _intra_pass(key_v, idx_v, k, NV, dir_all=False)
        return carry

    lax.fori_loop(5, 11, level, 0)


def _merge_prune(key_v, idx_v, pk_v, pi_v, h):
    iota = lax.iota(jnp.int32, L)

    @plsc.parallel_loop(0, NV, unroll=2)
    def _(v):
        base = v * L
        ak = key_v[pl.ds(base, L)]
        ai = idx_v[pl.ds(base, L)]
        rev = (CHUNK - 1 - base) - iota
        bk = plsc.load_gather(pk_v, [rev])
        bi = plsc.load_gather(pi_v, [rev])
        aw = _tot_gt(ak, ai, bk, bi)
        key_v[pl.ds(base, L)] = jnp.where(aw, ak, bk)
        idx_v[pl.ds(base, L)] = jnp.where(aw, ai, bi)

    _cross_stage(key_v, idx_v, 0, 512, CHUNK, dir_all=True)
    half = h * (CHUNK // 2)

    def cross(i, carry):
        _cross_stage(key_v, idx_v, 0, 256 >> i, CHUNK // 2, dir_all=True,
                     start=half)
        return carry

    lax.fori_loop(0, 5, cross, 0)
    _intra_pass(key_v, idx_v, 0, NV // 2, dir_all=True, start=half)


def _sc_body(sco_hbm, x_hbm, xv_hbm, xg_hbm, xvg_hbm, sg_hbm,
             key_v, idx_v, pk_v, pi_v, idxg_v, rows_v, rows2_v, sco_f,
             keys_sh, idx_sh, sem, sem2, sem3, sem4):
    c = lax.axis_index("c")
    s = lax.axis_index("s")
    g = s // 8
    q = s % 8
    b = c * 2 + g
    iota = lax.iota(jnp.int32, L)

    base_in = b * N + q * CHUNK
    pltpu.sync_copy(sco_hbm.at[pl.ds(base_in, CHUNK)], key_v)

    @plsc.parallel_loop(0, NV, unroll=2)
    def _(v):
        idx_v[pl.ds(v * L, L)] = (q * CHUNK + v * L) + iota

    _local_sort(key_v, idx_v)

    pltpu.sync_copy(key_v, keys_sh.at[s])
    pltpu.sync_copy(idx_v, idx_sh.at[s])
    plsc.subcore_barrier()

    half = CHUNK // 2

    def _round(r, carry):
        nmerge = 4 >> r
        active = q < 2 * nmerge
        m = q & (nmerge - 1)
        h = q >> (2 - r)

        @pl.when(active)
        def _read():
            slot = g * 8 + 2 * m
            pltpu.sync_copy(keys_sh.at[slot], key_v)
            pltpu.sync_copy(idx_sh.at[slot], idx_v)
            pltpu.sync_copy(keys_sh.at[slot + 1], pk_v)
            pltpu.sync_copy(idx_sh.at[slot + 1], pi_v)

        plsc.subcore_barrier()

        @pl.when(active)
        def _merge():
            _merge_prune(key_v, idx_v, pk_v, pi_v, h)
            hs = h * half
            pltpu.sync_copy(key_v.at[pl.ds(hs, half)],
                            keys_sh.at[g * 8 + m, pl.ds(hs, half)])
            pltpu.sync_copy(idx_v.at[pl.ds(hs, half)],
                            idx_sh.at[g * 8 + m, pl.ds(hs, half)])

        plsc.subcore_barrier()
        return carry

    lax.fori_loop(0, 3, _round, 0)

    half = CHUNK // 2

    @pl.when(q < 2)
    def _write_scores():
        hs = q * half

        @plsc.parallel_loop(0, NV // 2, unroll=2)
        def _(v):
            sco_f[pl.ds(v * L, L)] = plsc.bitcast(
                key_v[pl.ds(hs + v * L, L)], jnp.float32)

        pltpu.sync_copy(sco_f, sg_hbm.at[pl.ds(b * K + hs, half)])

    rows_per_tile = K // 8
    pltpu.sync_copy(idx_sh.at[g * 8, pl.ds(q * rows_per_tile, rows_per_tile)],
                    idxg_v)
    for t in range(rows_per_tile // L):
        idxg_v[pl.ds(t * L, L)] = idxg_v[pl.ds(t * L, L)] + b * N
    out_base = b * K + q * rows_per_tile
    g1 = pltpu.async_copy(x_hbm.at[idxg_v], rows_v, sem)
    g2 = pltpu.async_copy(xv_hbm.at[idxg_v], rows2_v, sem2)
    g1.wait()
    w1 = pltpu.async_copy(rows_v, xg_hbm.at[pl.ds(out_base, rows_per_tile)],
                          sem3)
    g2.wait()
    w2 = pltpu.async_copy(rows2_v,
                          xvg_hbm.at[pl.ds(out_base, rows_per_tile)], sem4)
    w1.wait()
    w2.wait()


_sc_topk = functools.partial(
    pl.kernel,
    out_type=(
        jax.ShapeDtypeStruct((B * K, D), jnp.float32),
        jax.ShapeDtypeStruct((B * K, D), jnp.float32),
        jax.ShapeDtypeStruct((B * K,), jnp.float32),
    ),
    mesh=plsc.VectorSubcoreMesh(core_axis_name="c", subcore_axis_name="s"),
    compiler_params=pltpu.CompilerParams(needs_layout_passes=False),
    scratch_types=[
        pltpu.VMEM((CHUNK,), jnp.int32),
        pltpu.VMEM((CHUNK,), jnp.int32),
        pltpu.VMEM((CHUNK,), jnp.int32),
        pltpu.VMEM((CHUNK,), jnp.int32),
        pltpu.VMEM((K // 8,), jnp.int32),
        pltpu.VMEM((K // 8, D), jnp.float32),
        pltpu.VMEM((K // 8, D), jnp.float32),
        pltpu.VMEM((CHUNK // 2,), jnp.float32),
        pltpu.VMEM_SHARED((NS, CHUNK), jnp.int32),
        pltpu.VMEM_SHARED((NS, CHUNK), jnp.int32),
        pltpu.SemaphoreType.DMA,
        pltpu.SemaphoreType.DMA,
        pltpu.SemaphoreType.DMA,
        pltpu.SemaphoreType.DMA,
    ],
)(_sc_body)


@jax.jit
def kernel(x, x_v, W1_w, W1_b, V_w, V_b):
    s = _scores(x, W1_w, W1_b, V_w, V_b)
    xg, xvg, sg = _sc_topk(s.reshape(B * N),
                           x.reshape(B * N, D),
                           x_v.reshape(B * N, D))
    return (xg.reshape(B, K, D), xvg.reshape(B, K, D),
            sg.reshape(B, K, 1))
